# Initial kernel scaffold; baseline (speedup 1.0000x reference)
import jax
import jax.numpy as jnp
from jax.experimental import pallas as pl


def kernel(x, edge_index, Wl1, bl1, Wr1, Wl2, bl2, Wr2, W3, b3):
    raise NotImplementedError("write your pallas kernel here")


# speedup vs baseline: 2.6302x; 2.6302x over previous
import functools

import jax
import jax.numpy as jnp
from jax import lax
from jax.experimental import pallas as pl
from jax.experimental.pallas import tpu as pltpu
from jax.experimental.pallas import tpu_sc as plsc

N = 10000
E = 160000
D_IN = 256
H1 = 512
H2 = 512

NS = 16
CH = 80
EPT = E // NS
NCHUNK = EPT // CH
ROWS_A = 624
TAIL = N - NS * ROWS_A
DSL = 128

_F32 = jnp.float32

_MESH = dict(core_axis_name="c", subcore_axis_name="s",
             num_cores=2, num_subcores=NS)


def _stripe_fill(buf, dst, r0, s):
  nfull = ROWS_A // CH
  rem = ROWS_A - nfull * CH
  for k in range(nfull):
    pltpu.sync_copy(buf, dst.at[pl.ds(r0 + k * CH, CH)])
  pltpu.sync_copy(buf.at[pl.ds(0, rem)], dst.at[pl.ds(r0 + nfull * CH, rem)])

  @pl.when(s == NS - 1)
  def _():
    pltpu.sync_copy(buf.at[pl.ds(0, TAIL)], dst.at[pl.ds(NS * ROWS_A, TAIL)])


def _stripe_out(src, buf, dst, r0, base, s):
  nfull = ROWS_A // CH
  rem = ROWS_A - nfull * CH
  for k in range(nfull):
    pltpu.sync_copy(src.at[pl.ds(r0 + k * CH, CH)], buf)
    pltpu.sync_copy(buf, dst.at[pl.ds(base + r0 + k * CH, CH)])
  pltpu.sync_copy(src.at[pl.ds(r0 + nfull * CH, rem)], buf.at[pl.ds(0, rem)])
  pltpu.sync_copy(buf.at[pl.ds(0, rem)],
                  dst.at[pl.ds(base + r0 + nfull * CH, rem)])

  @pl.when(s == NS - 1)
  def _():
    t0 = NS * ROWS_A
    pltpu.sync_copy(src.at[pl.ds(t0, TAIL)], buf.at[pl.ds(0, TAIL)])
    pltpu.sync_copy(buf.at[pl.ds(0, TAIL)], dst.at[pl.ds(base + t0, TAIL)])


@functools.cache
def _seg_sum_builder(interpret=False):
  scratch = (
      pltpu.VMEM_SHARED((N, DSL), _F32),
      pltpu.VMEM((CH,), jnp.int32),
      pltpu.VMEM((CH,), jnp.int32),
      pltpu.VMEM((CH, DSL), _F32),
      pltpu.SemaphoreType.DMA,
  )
  mesh = plsc.VectorSubcoreMesh(**_MESH)

  def body(xs_h, src_h, dst_h, z128_h, out_h, acc, srcv, dstv, rows, sem):
    c = lax.axis_index("c")
    s = lax.axis_index("s")
    r0 = s * ROWS_A

    pltpu.sync_copy(z128_h.at[pl.ds(0, CH)], rows)
    _stripe_fill(rows, acc, r0, s)
    plsc.subcore_barrier()

    base_src = c * E + s * EPT
    base_dst = s * EPT

    def chunk(j, carry):
      o = j * CH
      pltpu.sync_copy(src_h.at[pl.ds(base_src + o, CH)], srcv)
      pltpu.sync_copy(dst_h.at[pl.ds(base_dst + o, CH)], dstv)
      pltpu.async_copy(xs_h.at[srcv], rows, sem).wait()
      pltpu.sync_copy(rows, acc.at[dstv], add=True)
      return carry

    lax.fori_loop(0, NCHUNK, chunk, 0)
    plsc.subcore_barrier()

    _stripe_out(acc, rows, out_h, r0, c * N, s)

  return pl.kernel(body, out_type=jax.ShapeDtypeStruct((2 * N, DSL), _F32),
                   mesh=mesh, scratch_types=scratch, interpret=interpret)


@functools.cache
def _count_builder(interpret=False):
  scratch = (
      pltpu.VMEM_SHARED((N, DSL), _F32),
      pltpu.VMEM((CH,), jnp.int32),
      pltpu.VMEM((CH, DSL), _F32),
      pltpu.SemaphoreType.DMA,
  )
  mesh = plsc.VectorSubcoreMesh(**_MESH)

  def body(dst_h, z128_h, ones_h, out_h, cacc, dstv, buf, sem):
    c = lax.axis_index("c")
    s = lax.axis_index("s")
    r0 = s * ROWS_A

    pltpu.sync_copy(z128_h.at[pl.ds(0, CH)], buf)
    _stripe_fill(buf, cacc, r0, s)
    pltpu.sync_copy(ones_h, buf)
    plsc.subcore_barrier()

    base_dst = s * EPT

    def chunk(j, carry):
      pltpu.sync_copy(dst_h.at[pl.ds(base_dst + j * CH, CH)], dstv)
      pltpu.sync_copy(buf, cacc.at[dstv], add=True)
      return carry

    lax.fori_loop(0, NCHUNK, chunk, 0)
    plsc.subcore_barrier()

    _stripe_out(cacc, buf, out_h, r0, c * N, s)

  return pl.kernel(body, out_type=jax.ShapeDtypeStruct((2 * N, DSL), _F32),
                   mesh=mesh, scratch_types=scratch, interpret=interpret)


BN = 1000


def _tc_layer1_builder(interpret=False):
  grid = (N // BN,)

  def body(s0, s1, cnt, x, wl, bl, wr, o0, o1, o2, o3):
    summed = jnp.concatenate([s0[...], s1[...]], axis=1)
    scale = 1.0 / jnp.maximum(cnt[...], 1.0)
    mean = summed * scale
    h = lax.dot_general(mean, wl[...], (((1,), (1,)), ((), ())),
                        preferred_element_type=_F32)
    h = h + lax.dot_general(x[...], wr[...], (((1,), (1,)), ((), ())),
                            preferred_element_type=_F32)
    h = jnp.maximum(h + bl[...], 0.0)
    o0[...] = h[:, 0:128]
    o1[...] = h[:, 128:256]
    o2[...] = h[:, 256:384]
    o3[...] = h[:, 384:512]

  return pl.pallas_call(
      body,
      grid=grid,
      in_specs=[
          pl.BlockSpec((BN, DSL), lambda i: (i, 0)),
          pl.BlockSpec((BN, DSL), lambda i: (i, 0)),
          pl.BlockSpec((BN, 1), lambda i: (i, 0)),
          pl.BlockSpec((BN, D_IN), lambda i: (i, 0)),
          pl.BlockSpec((H1, D_IN), lambda i: (0, 0)),
          pl.BlockSpec((1, H1), lambda i: (0, 0)),
          pl.BlockSpec((H1, D_IN), lambda i: (0, 0)),
      ],
      out_specs=[pl.BlockSpec((BN, DSL), lambda i: (i, 0))] * 4,
      out_shape=[jax.ShapeDtypeStruct((N, DSL), _F32)] * 4,
      interpret=interpret,
  )


def _tc_layer2_builder(interpret=False):
  grid = (N // BN,)

  def body(s0, s1, s2, s3, cnt, h0, h1r, h2r, h3r, wl, bl, wr, w3, b3, o):
    summed = jnp.concatenate([s0[...], s1[...], s2[...], s3[...]], axis=1)
    hprev = jnp.concatenate([h0[...], h1r[...], h2r[...], h3r[...]], axis=1)
    scale = 1.0 / jnp.maximum(cnt[...], 1.0)
    mean = summed * scale
    g = lax.dot_general(mean, wl[...], (((1,), (1,)), ((), ())),
                        preferred_element_type=_F32)
    g = g + lax.dot_general(hprev, wr[...], (((1,), (1,)), ((), ())),
                            preferred_element_type=_F32)
    h = jnp.maximum(g + bl[...], 0.0)
    out = lax.dot_general(h, w3[...], (((1,), (1,)), ((), ())),
                          preferred_element_type=_F32)
    o[...] = out + b3[0, 0]

  return pl.pallas_call(
      body,
      grid=grid,
      in_specs=(
          [pl.BlockSpec((BN, DSL), lambda i: (i, 0))] * 4
          + [pl.BlockSpec((BN, 1), lambda i: (i, 0))]
          + [pl.BlockSpec((BN, DSL), lambda i: (i, 0))] * 4
          + [
              pl.BlockSpec((H2, H1), lambda i: (0, 0)),
              pl.BlockSpec((1, H2), lambda i: (0, 0)),
              pl.BlockSpec((H2, H1), lambda i: (0, 0)),
              pl.BlockSpec((DSL, H2), lambda i: (0, 0)),
              pl.BlockSpec(memory_space=pltpu.SMEM),
          ]
      ),
      out_specs=[pl.BlockSpec((BN, DSL), lambda i: (i, 0))],
      out_shape=[jax.ShapeDtypeStruct((N, DSL), _F32)],
      interpret=interpret,
  )


_tc_layer1 = _tc_layer1_builder()
_tc_layer2 = _tc_layer2_builder()


@jax.jit
def _run(x, src2, dst, Wl1, bl1, Wr1, Wl2, bl2, Wr2, W3, b3):
  _seg_sum = _seg_sum_builder()
  _count = _count_builder()
  z128 = jnp.zeros((N, DSL), _F32)
  ones = jnp.ones((CH, DSL), _F32)
  xs = jnp.concatenate([x[:, :DSL], x[:, DSL:]], axis=0)
  cnt128 = _count(dst, z128, ones)
  cnt = cnt128[:N, :1]
  sum1 = _seg_sum(xs, src2, dst, z128)
  h0, h1, h2, h3 = _tc_layer1(sum1[:N], sum1[N:], cnt, x,
                              Wl1, bl1.reshape(1, H1), Wr1)
  hs01 = jnp.concatenate([h0, h1], axis=0)
  hs23 = jnp.concatenate([h2, h3], axis=0)
  sum2a = _seg_sum(hs01, src2, dst, z128)
  sum2b = _seg_sum(hs23, src2, dst, z128)
  w3p = jnp.zeros((DSL, H2), _F32).at[0].set(W3[0])
  (out,) = _tc_layer2(sum2a[:N], sum2a[N:], sum2b[:N], sum2b[N:],
                      cnt, h0, h1, h2, h3,
                      Wl2, bl2.reshape(1, H2), Wr2,
                      w3p, b3.reshape(1, 1))
  return out[:, :1]


def kernel(x, edge_index, Wl1, bl1, Wr1, Wl2, bl2, Wr2, W3, b3):
  src = edge_index[0].astype(jnp.int32)
  dst = edge_index[1].astype(jnp.int32)
  src2 = jnp.concatenate([src, src + N])
  return _run(x, src2, dst, Wl1, bl1, Wr1, Wl2, bl2, Wr2, W3, b3)


# speedup vs baseline: 5.7891x; 2.2010x over previous
import functools

import jax
import jax.numpy as jnp
from jax import lax
from jax.experimental import pallas as pl
from jax.experimental.pallas import tpu as pltpu
from jax.experimental.pallas import tpu_sc as plsc

N = 10000
E = 160000
D_IN = 256
H1 = 512
H2 = 512

NS = 16
CH = 80
EPT = E // NS
NCHUNK = EPT // CH
ROWS_A = 624
TAIL = N - NS * ROWS_A
DSL = 128

_F32 = jnp.float32

_MESH = dict(core_axis_name="c", subcore_axis_name="s",
             num_cores=2, num_subcores=NS)


def _stripe_fill(buf, dst, r0, s):
  nfull = ROWS_A // CH
  rem = ROWS_A - nfull * CH
  for k in range(nfull):
    pltpu.sync_copy(buf, dst.at[pl.ds(r0 + k * CH, CH)])
  pltpu.sync_copy(buf.at[pl.ds(0, rem)], dst.at[pl.ds(r0 + nfull * CH, rem)])

  @pl.when(s == NS - 1)
  def _():
    pltpu.sync_copy(buf.at[pl.ds(0, TAIL)], dst.at[pl.ds(NS * ROWS_A, TAIL)])


def _stripe_out(src, buf, dst, r0, base, s):
  nfull = ROWS_A // CH
  rem = ROWS_A - nfull * CH
  for k in range(nfull):
    pltpu.sync_copy(src.at[pl.ds(r0 + k * CH, CH)], buf)
    pltpu.sync_copy(buf, dst.at[pl.ds(base + r0 + k * CH, CH)])
  pltpu.sync_copy(src.at[pl.ds(r0 + nfull * CH, rem)], buf.at[pl.ds(0, rem)])
  pltpu.sync_copy(buf.at[pl.ds(0, rem)],
                  dst.at[pl.ds(base + r0 + nfull * CH, rem)])

  @pl.when(s == NS - 1)
  def _():
    t0 = NS * ROWS_A
    pltpu.sync_copy(src.at[pl.ds(t0, TAIL)], buf.at[pl.ds(0, TAIL)])
    pltpu.sync_copy(buf.at[pl.ds(0, TAIL)], dst.at[pl.ds(base + t0, TAIL)])


@functools.cache
def _seg_sum_builder(interpret=False):
  scratch = (
      pltpu.VMEM_SHARED((N, DSL), _F32),
      pltpu.VMEM((EPT,), jnp.int32),
      pltpu.VMEM((NCHUNK, CH), jnp.int32),
      pltpu.VMEM((CH, DSL), _F32),
      pltpu.VMEM((CH, DSL), _F32),
      pltpu.SemaphoreType.DMA,
      pltpu.SemaphoreType.DMA,
  )
  mesh = plsc.VectorSubcoreMesh(**_MESH)

  def body(xs_h, src_h, dst3_h, z128_h, out_h,
           acc, srca, dsta, rows0, rows1, sem0, sem1):
    c = lax.axis_index("c")
    s = lax.axis_index("s")
    r0 = s * ROWS_A

    pltpu.sync_copy(z128_h.at[pl.ds(0, CH)], rows0)
    _stripe_fill(rows0, acc, r0, s)
    pltpu.sync_copy(src_h.at[pl.ds(c * E + s * EPT, EPT)], srca)
    pltpu.sync_copy(dst3_h.at[s], dsta)
    plsc.subcore_barrier()

    def gidx(j):
      return srca.at[pl.ds(j * CH, CH)]

    pltpu.async_copy(xs_h.at[gidx(0)], rows0, sem0)

    def pair(p, carry):
      j0 = 2 * p
      pltpu.async_copy(xs_h.at[gidx(j0 + 1)], rows1, sem1)
      pltpu.make_async_copy(xs_h.at[gidx(j0)], rows0, sem0).wait()
      pltpu.sync_copy(rows0, acc.at[dsta.at[j0]], add=True)
      pltpu.async_copy(xs_h.at[gidx(j0 + 2)], rows0, sem0)
      pltpu.make_async_copy(xs_h.at[gidx(j0 + 1)], rows1, sem1).wait()
      pltpu.sync_copy(rows1, acc.at[dsta.at[j0 + 1]], add=True)
      return carry

    lax.fori_loop(0, NCHUNK // 2, pair, 0)
    pltpu.make_async_copy(xs_h.at[gidx(NCHUNK - 1)], rows0, sem0).wait()
    pltpu.sync_copy(rows0, acc.at[dsta.at[NCHUNK - 1]], add=True)
    plsc.subcore_barrier()

    _stripe_out(acc, rows0, out_h, r0, c * N, s)

  return pl.kernel(body, out_type=jax.ShapeDtypeStruct((2 * N, DSL), _F32),
                   mesh=mesh, scratch_types=scratch, interpret=interpret)


NCH0 = 63


@functools.cache
def _count_builder(interpret=False):
  scratch = (
      pltpu.VMEM_SHARED((N, DSL), _F32),
      pltpu.VMEM((NCHUNK, CH), jnp.int32),
      pltpu.VMEM((CH, DSL), _F32),
      pltpu.SemaphoreType.DMA,
  )
  mesh = plsc.VectorSubcoreMesh(**_MESH)

  def body(dst3_h, z128_h, ones_h, out_h, cacc, dsta, buf, sem):
    c = lax.axis_index("c")
    s = lax.axis_index("s")
    r0 = s * ROWS_A

    pltpu.sync_copy(z128_h.at[pl.ds(0, CH)], buf)
    _stripe_fill(buf, cacc, r0, s)
    pltpu.sync_copy(dst3_h.at[s], dsta)
    pltpu.sync_copy(ones_h, buf)
    plsc.subcore_barrier()

    cbase = c * NCH0

    def chunk(j, carry):
      pltpu.sync_copy(buf, cacc.at[dsta.at[cbase + j]], add=True)
      return carry

    lax.fori_loop(0, NCH0 - c, chunk, 0)
    plsc.subcore_barrier()

    _stripe_out(cacc, buf, out_h, r0, c * N, s)

  return pl.kernel(body, out_type=jax.ShapeDtypeStruct((2 * N, DSL), _F32),
                   mesh=mesh, scratch_types=scratch, interpret=interpret)


BN = 1000


def _tc_layer1_builder(interpret=False):
  grid = (N // BN,)

  def body(s0, s1, cnt0, cnt1, x, wl, bl, wr, o0, o1, o2, o3):
    summed = jnp.concatenate([s0[...], s1[...]], axis=1)
    scale = 1.0 / jnp.maximum(cnt0[...] + cnt1[...], 1.0)
    mean = summed * scale
    h = lax.dot_general(mean, wl[...], (((1,), (1,)), ((), ())),
                        preferred_element_type=_F32)
    h = h + lax.dot_general(x[...], wr[...], (((1,), (1,)), ((), ())),
                            preferred_element_type=_F32)
    h = jnp.maximum(h + bl[...], 0.0)
    o0[...] = h[:, 0:128]
    o1[...] = h[:, 128:256]
    o2[...] = h[:, 256:384]
    o3[...] = h[:, 384:512]

  return pl.pallas_call(
      body,
      grid=grid,
      in_specs=[
          pl.BlockSpec((BN, DSL), lambda i: (i, 0)),
          pl.BlockSpec((BN, DSL), lambda i: (i, 0)),
          pl.BlockSpec((BN, 1), lambda i: (i, 0)),
          pl.BlockSpec((BN, 1), lambda i: (i, 0)),
          pl.BlockSpec((BN, D_IN), lambda i: (i, 0)),
          pl.BlockSpec((H1, D_IN), lambda i: (0, 0)),
          pl.BlockSpec((1, H1), lambda i: (0, 0)),
          pl.BlockSpec((H1, D_IN), lambda i: (0, 0)),
      ],
      out_specs=[pl.BlockSpec((BN, DSL), lambda i: (i, 0))] * 4,
      out_shape=[jax.ShapeDtypeStruct((N, DSL), _F32)] * 4,
      interpret=interpret,
  )


def _tc_layer2_builder(interpret=False):
  grid = (N // BN,)

  def body(s0, s1, s2, s3, cnt0, cnt1, h0, h1r, h2r, h3r, wl, bl, wr, w3, b3, o):
    summed = jnp.concatenate([s0[...], s1[...], s2[...], s3[...]], axis=1)
    hprev = jnp.concatenate([h0[...], h1r[...], h2r[...], h3r[...]], axis=1)
    scale = 1.0 / jnp.maximum(cnt0[...] + cnt1[...], 1.0)
    mean = summed * scale
    g = lax.dot_general(mean, wl[...], (((1,), (1,)), ((), ())),
                        preferred_element_type=_F32)
    g = g + lax.dot_general(hprev, wr[...], (((1,), (1,)), ((), ())),
                            preferred_element_type=_F32)
    h = jnp.maximum(g + bl[...], 0.0)
    out = lax.dot_general(h, w3[...], (((1,), (1,)), ((), ())),
                          preferred_element_type=_F32)
    o[...] = out + b3[0, 0]

  return pl.pallas_call(
      body,
      grid=grid,
      in_specs=(
          [pl.BlockSpec((BN, DSL), lambda i: (i, 0))] * 4
          + [pl.BlockSpec((BN, 1), lambda i: (i, 0))] * 2
          + [pl.BlockSpec((BN, DSL), lambda i: (i, 0))] * 4
          + [
              pl.BlockSpec((H2, H1), lambda i: (0, 0)),
              pl.BlockSpec((1, H2), lambda i: (0, 0)),
              pl.BlockSpec((H2, H1), lambda i: (0, 0)),
              pl.BlockSpec((DSL, H2), lambda i: (0, 0)),
              pl.BlockSpec(memory_space=pltpu.SMEM),
          ]
      ),
      out_specs=[pl.BlockSpec((BN, DSL), lambda i: (i, 0))],
      out_shape=[jax.ShapeDtypeStruct((N, DSL), _F32)],
      interpret=interpret,
  )


_tc_layer1 = _tc_layer1_builder()
_tc_layer2 = _tc_layer2_builder()


@jax.jit
def _run(x, src2, dst3, Wl1, bl1, Wr1, Wl2, bl2, Wr2, W3, b3):
  _seg_sum = _seg_sum_builder()
  _count = _count_builder()
  z128 = jnp.zeros((N, DSL), _F32)
  ones = jnp.ones((CH, DSL), _F32)
  xs = jnp.concatenate([x[:, :DSL], x[:, DSL:]], axis=0)
  cnt128 = _count(dst3, z128, ones)
  cnt0 = cnt128[:N, :1]
  cnt1 = cnt128[N:, :1]
  sum1 = _seg_sum(xs, src2, dst3, z128)
  h0, h1, h2, h3 = _tc_layer1(sum1[:N], sum1[N:], cnt0, cnt1, x,
                              Wl1, bl1.reshape(1, H1), Wr1)
  hs01 = jnp.concatenate([h0, h1], axis=0)
  hs23 = jnp.concatenate([h2, h3], axis=0)
  sum2a = _seg_sum(hs01, src2, dst3, z128)
  sum2b = _seg_sum(hs23, src2, dst3, z128)
  w3p = jnp.zeros((DSL, H2), _F32).at[0].set(W3[0])
  (out,) = _tc_layer2(sum2a[:N], sum2a[N:], sum2b[:N], sum2b[N:],
                      cnt0, cnt1, h0, h1, h2, h3,
                      Wl2, bl2.reshape(1, H2), Wr2,
                      w3p, b3.reshape(1, 1))
  return out[:, :1]


def kernel(x, edge_index, Wl1, bl1, Wr1, Wl2, bl2, Wr2, W3, b3):
  src = edge_index[0].astype(jnp.int32)
  dst = edge_index[1].astype(jnp.int32)
  src2 = jnp.concatenate([src, src + N])
  dst3 = dst.reshape(NS, NCHUNK, CH)
  return _run(x, src2, dst3, Wl1, bl1, Wr1, Wl2, bl2, Wr2, W3, b3)
